# baseline (speedup 1.0000x reference)
import functools

import jax
import jax.numpy as jnp
from jax import lax
from jax.experimental import pallas as pl
from jax.experimental.pallas import tpu as pltpu
from jax.experimental.pallas import tpu_sc as plsc

S = 2048
H = 1024
DFF = 2048
E = 8
EPAD = 128
TOPK = 2
AUX_COEF = 0.01
Z_COEF = 0.001

BLK = 256
NB = S * TOPK // BLK + E
NROWS = NB * BLK
CB = 256

NC = 2
NS = 16
NT = NC * NS
RPT = NROWS // NT
TPT = S // NT
GCH = 64


def _routing_body(x_ref, wr_ref, meta_ref, counts_ref, aux_ref, c_ref):
    x = x_ref[...]
    wr = wr_ref[...]
    lt = lax.dot_general(wr, x, (((1,), (1,)), ((), ())),
                         preferred_element_type=jnp.float32)
    row = lax.broadcasted_iota(jnp.int32, (EPAD, S), 0)
    valid = row < E
    neg = jnp.float32(-1e30)
    lt = jnp.where(valid, lt, neg)

    lmax = jnp.max(lt, axis=0, keepdims=True)
    ex = jnp.exp(lt - lmax)
    ssum = jnp.sum(ex, axis=0, keepdims=True)
    probs = ex / ssum

    m1 = jnp.max(probs, axis=0, keepdims=True)
    a1 = jnp.min(jnp.where(probs == m1, row, EPAD), axis=0, keepdims=True)
    probs2 = jnp.where(row == a1, -1.0, probs)
    m2 = jnp.max(probs2, axis=0, keepdims=True)
    a2 = jnp.min(jnp.where(probs2 == m2, row, EPAD), axis=0, keepdims=True)
    den = m1 + m2
    g1 = m1 / den
    g2 = m2 / den

    oh = ((row == a1) | (row == a2)).astype(jnp.float32)
    tri = (lax.broadcasted_iota(jnp.int32, (CB, CB), 0)
           <= lax.broadcasted_iota(jnp.int32, (CB, CB), 1)).astype(jnp.float32)
    carry = jnp.zeros((EPAD, 1), jnp.float32)
    for i in range(S // CB):
        ohb = oh[:, i * CB:(i + 1) * CB]
        c_ref[:, i * CB:(i + 1) * CB] = carry + lax.dot_general(
            ohb, tri, (((1,), (0,)), ((), ())),
            preferred_element_type=jnp.float32)
        carry = carry + jnp.sum(ohb, axis=1, keepdims=True)
    counts = carry

    nb = jnp.floor((counts + jnp.float32(BLK - 1)) * jnp.float32(1.0 / BLK))
    tril = (lax.broadcasted_iota(jnp.int32, (EPAD, EPAD), 1)
            <= lax.broadcasted_iota(jnp.int32, (EPAD, EPAD), 0)).astype(jnp.float32)
    cumnb = lax.dot_general(tril, nb, (((1,), (0,)), ((), ())),
                            preferred_element_type=jnp.float32)
    base = (cumnb - nb) * jnp.float32(BLK)

    c_all = c_ref[...]
    sel1 = jnp.sum(jnp.where(row == a1, c_all, 0.0), axis=0, keepdims=True)
    sel2 = jnp.sum(jnp.where(row == a2, c_all, 0.0), axis=0, keepdims=True)
    base1 = jnp.sum(jnp.where(row == a1, base, 0.0), axis=0, keepdims=True)
    base2 = jnp.sum(jnp.where(row == a2, base, 0.0), axis=0, keepdims=True)
    dst1 = base1 + sel1 - 1.0
    dst2 = base2 + sel2 - 1.0

    meta_ref[0:1, :] = a1.astype(jnp.float32)
    meta_ref[1:2, :] = a2.astype(jnp.float32)
    meta_ref[2:3, :] = g1
    meta_ref[3:4, :] = g2
    meta_ref[4:5, :] = dst1
    meta_ref[5:6, :] = dst2
    meta_ref[6:7, :] = jnp.zeros((1, S), jnp.float32)
    meta_ref[7:8, :] = jnp.zeros((1, S), jnp.float32)
    counts_ref[...] = counts

    fraction = counts / jnp.float32(S * TOPK)
    mean_prob = jnp.sum(probs, axis=1, keepdims=True) / jnp.float32(S)
    lb = jnp.float32(E) * jnp.sum(fraction * mean_prob)
    lse = jnp.log(ssum) + lmax
    z = jnp.sum(lse * lse) / jnp.float32(S)
    aux_ref[...] = jnp.reshape(AUX_COEF * lb + Z_COEF * z, (1, 1))


def _routing(x2d, wr_pad):
    return pl.pallas_call(
        _routing_body,
        out_shape=(
            jax.ShapeDtypeStruct((8, S), jnp.float32),
            jax.ShapeDtypeStruct((EPAD, 1), jnp.float32),
            jax.ShapeDtypeStruct((1, 1), jnp.float32),
        ),
        in_specs=[
            pl.BlockSpec((S, H), lambda: (0, 0)),
            pl.BlockSpec((EPAD, H), lambda: (0, 0)),
        ],
        out_specs=(
            pl.BlockSpec((8, S), lambda: (0, 0)),
            pl.BlockSpec((EPAD, 1), lambda: (0, 0)),
            pl.BlockSpec((1, 1), lambda: (0, 0)),
        ),
        scratch_shapes=[pltpu.VMEM((EPAD, S), jnp.float32)],
    )(x2d, wr_pad)


def _dispatch_kernel(d1_hbm, d2_hbm, g1_hbm, g2_hbm, x_hbm, xg_hbm, gs_hbm,
                     d1v, d2v, g1v, g2v, srcl, gl, xbuf, sem):
    wid = lax.axis_index("s") * NC + lax.axis_index("c")
    lo = wid * RPT

    pltpu.sync_copy(d1_hbm, d1v)
    pltpu.sync_copy(d2_hbm, d2v)
    pltpu.sync_copy(g1_hbm, g1v)
    pltpu.sync_copy(g2_hbm, g2v)

    zi = jnp.zeros((16,), jnp.int32)
    zf = jnp.zeros((16,), jnp.float32)
    for k in range(RPT // 16):
        srcl[pl.ds(k * 16, 16)] = zi
        gl[pl.ds(k * 16, 16)] = zf

    lane = lax.broadcasted_iota(jnp.int32, (16,), 0)

    def body(c, _):
        toks = lane + c * 16
        d1c = d1v[pl.ds(c * 16, 16)].astype(jnp.int32)
        p1 = d1c - lo
        m1 = (p1 >= 0) & (p1 < RPT)
        plsc.store_scatter(srcl, [p1], toks, mask=m1)
        plsc.store_scatter(gl, [p1], g1v[pl.ds(c * 16, 16)], mask=m1)
        d2c = d2v[pl.ds(c * 16, 16)].astype(jnp.int32)
        p2 = d2c - lo
        m2 = (p2 >= 0) & (p2 < RPT)
        plsc.store_scatter(srcl, [p2], toks, mask=m2)
        plsc.store_scatter(gl, [p2], g2v[pl.ds(c * 16, 16)], mask=m2)
        return ()

    lax.fori_loop(0, S // 16, body, ())

    pltpu.sync_copy(gl, gs_hbm.at[pl.ds(lo, RPT)])
    for r in range(RPT // GCH):
        pltpu.async_copy(x_hbm.at[srcl.at[pl.ds(r * GCH, GCH)]], xbuf,
                         sem).wait()
        pltpu.sync_copy(xbuf, xg_hbm.at[pl.ds(lo + r * GCH, GCH)])


def _dispatch(d1, d2, g1, g2, x2d):
    mesh = plsc.VectorSubcoreMesh(core_axis_name="c", subcore_axis_name="s")
    return pl.kernel(
        _dispatch_kernel,
        mesh=mesh,
        compiler_params=pltpu.CompilerParams(needs_layout_passes=False),
        out_type=(
            jax.ShapeDtypeStruct((NROWS, H), jnp.float32),
            jax.ShapeDtypeStruct((NROWS,), jnp.float32),
        ),
        scratch_types=[
            pltpu.VMEM((S,), jnp.float32),
            pltpu.VMEM((S,), jnp.float32),
            pltpu.VMEM((S,), jnp.float32),
            pltpu.VMEM((S,), jnp.float32),
            pltpu.VMEM((RPT,), jnp.int32),
            pltpu.VMEM((RPT,), jnp.float32),
            pltpu.VMEM((GCH, H), jnp.float32),
            pltpu.SemaphoreType.DMA,
        ],
    )(d1, d2, g1, g2, x2d)


def _ffn_body(exp_ref, val_ref, xg_ref, gs_ref, w1_ref, b1_ref, w2_ref,
              b2_ref, out_ref):
    b = pl.program_id(0)

    @pl.when(val_ref[b] == 1)
    def _():
        xb = xg_ref[...]
        h = lax.dot_general(xb, w1_ref[0], (((1,), (1,)), ((), ())),
                            preferred_element_type=jnp.float32)
        h = h + b1_ref[0]
        h = 0.5 * h * (1.0 + lax.erf(h * jnp.float32(0.7071067811865476)))
        y = lax.dot_general(h, w2_ref[0], (((1,), (1,)), ((), ())),
                            preferred_element_type=jnp.float32)
        y = y + b2_ref[0]
        out_ref[...] = y * gs_ref[...]


def _ffn(exp_id, valid, xg, gs2d, W1, b1, W2, b2):
    grid_spec = pltpu.PrefetchScalarGridSpec(
        num_scalar_prefetch=2,
        grid=(NB,),
        in_specs=[
            pl.BlockSpec((BLK, H), lambda b, eref, vref: (b, 0)),
            pl.BlockSpec((BLK, 1), lambda b, eref, vref: (b, 0)),
            pl.BlockSpec((1, DFF, H), lambda b, eref, vref: (eref[b], 0, 0)),
            pl.BlockSpec((1, 1, DFF), lambda b, eref, vref: (eref[b], 0, 0)),
            pl.BlockSpec((1, H, DFF), lambda b, eref, vref: (eref[b], 0, 0)),
            pl.BlockSpec((1, 1, H), lambda b, eref, vref: (eref[b], 0, 0)),
        ],
        out_specs=pl.BlockSpec((BLK, H), lambda b, eref, vref: (b, 0)),
    )
    return pl.pallas_call(
        _ffn_body,
        grid_spec=grid_spec,
        out_shape=jax.ShapeDtypeStruct((NROWS, H), jnp.float32),
    )(exp_id, valid, xg, gs2d, W1, b1.reshape(E, 1, DFF), W2,
      b2.reshape(E, 1, H))


def _combine_kernel(d1_hbm, d2_hbm, ybuf_hbm, yab_hbm, dbuf, idx, rows, sem):
    wid = lax.axis_index("s") * NC + lax.axis_index("c")
    t0 = wid * TPT

    pltpu.sync_copy(d1_hbm.at[pl.ds(t0, TPT)], dbuf)
    for k in range(TPT // 16):
        idx[pl.ds(k * 16, 16)] = dbuf[pl.ds(k * 16, 16)].astype(jnp.int32)
    pltpu.async_copy(ybuf_hbm.at[idx], rows, sem).wait()
    pltpu.sync_copy(rows, yab_hbm.at[pl.ds(t0, TPT)])

    pltpu.sync_copy(d2_hbm.at[pl.ds(t0, TPT)], dbuf)
    for k in range(TPT // 16):
        idx[pl.ds(k * 16, 16)] = dbuf[pl.ds(k * 16, 16)].astype(jnp.int32)
    pltpu.async_copy(ybuf_hbm.at[idx], rows, sem).wait()
    pltpu.sync_copy(rows, yab_hbm.at[pl.ds(S + t0, TPT)])


def _combine(d1, d2, ybuf):
    mesh = plsc.VectorSubcoreMesh(core_axis_name="c", subcore_axis_name="s")
    return pl.kernel(
        _combine_kernel,
        mesh=mesh,
        compiler_params=pltpu.CompilerParams(needs_layout_passes=False),
        out_type=jax.ShapeDtypeStruct((2 * S, H), jnp.float32),
        scratch_types=[
            pltpu.VMEM((TPT,), jnp.float32),
            pltpu.VMEM((TPT,), jnp.int32),
            pltpu.VMEM((TPT, H), jnp.float32),
            pltpu.SemaphoreType.DMA,
        ],
    )(d1, d2, ybuf)


def _add_body(a_ref, b_ref, out_ref):
    out_ref[...] = a_ref[...] + b_ref[...]


def _add(yab):
    nsb = 4
    sb = S // nsb
    return pl.pallas_call(
        _add_body,
        grid=(nsb,),
        out_shape=jax.ShapeDtypeStruct((S, H), jnp.float32),
        in_specs=[
            pl.BlockSpec((sb, H), lambda i: (i, 0)),
            pl.BlockSpec((sb, H), lambda i: (i + nsb, 0)),
        ],
        out_specs=pl.BlockSpec((sb, H), lambda i: (i, 0)),
    )(yab, yab)


@jax.jit
def _moe(x, Wr, W1, b1, W2, b2):
    x2d = x.reshape(S, H)
    wr_pad = jnp.zeros((EPAD, H), jnp.float32).at[:E].set(Wr)

    meta, counts_col, aux = _routing(x2d, wr_pad)
    d1 = meta[4]
    d2 = meta[5]
    g1 = meta[2]
    g2 = meta[3]

    counts = counts_col[:E, 0].astype(jnp.int32)
    nb = (counts + (BLK - 1)) // BLK
    cum = jnp.cumsum(nb)
    barange = jnp.arange(NB, dtype=jnp.int32)
    exp_id = jnp.minimum(
        jnp.sum((barange[:, None] >= cum[None, :]).astype(jnp.int32), axis=1),
        E - 1)
    valid = (barange < cum[-1]).astype(jnp.int32)

    xg, gs = _dispatch(d1, d2, g1, g2, x2d)
    ybuf = _ffn(exp_id, valid, xg, gs.reshape(NROWS, 1), W1, b1, W2, b2)
    yab = _combine(d1, d2, ybuf)
    out = _add(yab)
    return out.reshape(1, S, H), aux[0, 0]


def kernel(x, Wr, W1, b1, W2, b2):
    return _moe(x, Wr, W1, b1, W2, b2)


# speedup vs baseline: 1.7160x; 1.7160x over previous
import jax
import jax.numpy as jnp
from jax import lax
from jax.experimental import pallas as pl
from jax.experimental.pallas import tpu as pltpu
from jax.experimental.pallas import tpu_sc as plsc

S = 2048
H = 1024
DFF = 2048
E = 8
EPAD = 128
TOPK = 2
AUX_COEF = 0.01
Z_COEF = 0.001

BLK = 256
NB = S * TOPK // BLK + E
NROWS = NB * BLK
RB = 256

NC = 2
NS = 16
NT = NC * NS
TPT = S // NT


def _routing_body(x_ref, wr_ref, d1_ref, d2_ref, g1_ref, g2_ref,
                  counts_ref, aux_ref, c_ref):
    x = x_ref[...]
    wr = wr_ref[...]
    logits = lax.dot_general(x, wr, (((1,), (1,)), ((), ())),
                             preferred_element_type=jnp.float32)
    lane = lax.broadcasted_iota(jnp.int32, (S, EPAD), 1)
    valid = lane < E
    neg = jnp.float32(-1e30)
    logits = jnp.where(valid, logits, neg)

    lmax = jnp.max(logits, axis=1, keepdims=True)
    ex = jnp.exp(logits - lmax)
    ssum = jnp.sum(ex, axis=1, keepdims=True)
    probs = ex / ssum

    m1 = jnp.max(probs, axis=1, keepdims=True)
    a1 = jnp.min(jnp.where(probs == m1, lane, EPAD), axis=1, keepdims=True)
    probs2 = jnp.where(lane == a1, -1.0, probs)
    m2 = jnp.max(probs2, axis=1, keepdims=True)
    a2 = jnp.min(jnp.where(probs2 == m2, lane, EPAD), axis=1, keepdims=True)
    den = m1 + m2
    g1_ref[...] = m1 / den
    g2_ref[...] = m2 / den

    oh = ((lane == a1) | (lane == a2)).astype(jnp.float32)
    tril = (lax.broadcasted_iota(jnp.int32, (RB, RB), 1)
            <= lax.broadcasted_iota(jnp.int32, (RB, RB), 0)).astype(jnp.float32)
    carry = jnp.zeros((1, EPAD), jnp.float32)
    for i in range(S // RB):
        ohb = oh[i * RB:(i + 1) * RB, :]
        c_ref[i * RB:(i + 1) * RB, :] = carry + lax.dot_general(
            tril, ohb, (((1,), (0,)), ((), ())),
            preferred_element_type=jnp.float32)
        carry = carry + jnp.sum(ohb, axis=0, keepdims=True)
    counts = carry
    counts_ref[...] = counts

    nb = jnp.floor((counts + jnp.float32(BLK - 1)) * jnp.float32(1.0 / BLK))
    triu = (lax.broadcasted_iota(jnp.int32, (EPAD, EPAD), 0)
            <= lax.broadcasted_iota(jnp.int32, (EPAD, EPAD), 1)).astype(jnp.float32)
    cumnb = lax.dot_general(nb, triu, (((1,), (0,)), ((), ())),
                            preferred_element_type=jnp.float32)
    base = (cumnb - nb) * jnp.float32(BLK)

    c_all = c_ref[...]
    sel1 = jnp.sum(jnp.where(lane == a1, c_all, 0.0), axis=1, keepdims=True)
    sel2 = jnp.sum(jnp.where(lane == a2, c_all, 0.0), axis=1, keepdims=True)
    base1 = jnp.sum(jnp.where(lane == a1, base, 0.0), axis=1, keepdims=True)
    base2 = jnp.sum(jnp.where(lane == a2, base, 0.0), axis=1, keepdims=True)
    d1_ref[...] = (base1 + sel1 - 1.0).astype(jnp.int32)
    d2_ref[...] = (base2 + sel2 - 1.0).astype(jnp.int32)

    fraction = counts / jnp.float32(S * TOPK)
    mean_prob = jnp.sum(probs, axis=0, keepdims=True) / jnp.float32(S)
    lb = jnp.float32(E) * jnp.sum(fraction * mean_prob)
    lse = jnp.log(ssum) + lmax
    z = jnp.sum(lse * lse) / jnp.float32(S)
    aux_ref[...] = jnp.reshape(AUX_COEF * lb + Z_COEF * z, (1, 1))


def _routing(x2d, wr_pad):
    return pl.pallas_call(
        _routing_body,
        out_shape=(
            jax.ShapeDtypeStruct((S, 1), jnp.int32),
            jax.ShapeDtypeStruct((S, 1), jnp.int32),
            jax.ShapeDtypeStruct((S, 1), jnp.float32),
            jax.ShapeDtypeStruct((S, 1), jnp.float32),
            jax.ShapeDtypeStruct((1, EPAD), jnp.float32),
            jax.ShapeDtypeStruct((1, 1), jnp.float32),
        ),
        in_specs=[
            pl.BlockSpec((S, H), lambda: (0, 0)),
            pl.BlockSpec((EPAD, H), lambda: (0, 0)),
        ],
        out_specs=(
            pl.BlockSpec((S, 1), lambda: (0, 0)),
            pl.BlockSpec((S, 1), lambda: (0, 0)),
            pl.BlockSpec((S, 1), lambda: (0, 0)),
            pl.BlockSpec((S, 1), lambda: (0, 0)),
            pl.BlockSpec((1, EPAD), lambda: (0, 0)),
            pl.BlockSpec((1, 1), lambda: (0, 0)),
        ),
        scratch_shapes=[pltpu.VMEM((S, EPAD), jnp.float32)],
    )(x2d, wr_pad)


def _dispatch_kernel(d1_hbm, d2_hbm, x_hbm, xg_hbm, idxr, xbuf, sem):
    wid = lax.axis_index("s") * NC + lax.axis_index("c")
    t0 = wid * TPT

    pltpu.sync_copy(x_hbm.at[pl.ds(t0, TPT)], xbuf)
    pltpu.sync_copy(d1_hbm.at[pl.ds(t0, TPT)], idxr)
    pltpu.async_copy(xbuf, xg_hbm.at[idxr], sem).wait()
    pltpu.sync_copy(d2_hbm.at[pl.ds(t0, TPT)], idxr)
    pltpu.async_copy(xbuf, xg_hbm.at[idxr], sem).wait()


def _dispatch(d1, d2, x2d):
    mesh = plsc.VectorSubcoreMesh(core_axis_name="c", subcore_axis_name="s")
    return pl.kernel(
        _dispatch_kernel,
        mesh=mesh,
        compiler_params=pltpu.CompilerParams(needs_layout_passes=False),
        out_type=jax.ShapeDtypeStruct((NROWS, H), jnp.float32),
        scratch_types=[
            pltpu.VMEM((TPT,), jnp.int32),
            pltpu.VMEM((TPT, H), jnp.float32),
            pltpu.SemaphoreType.DMA,
        ],
    )(d1, d2, x2d)


def _ffn_body(exp_ref, val_ref, xg_ref, w1_ref, b1_ref, w2_ref, b2_ref,
              out_ref):
    b = pl.program_id(0)

    @pl.when(val_ref[b] == 1)
    def _():
        xb = xg_ref[...]
        h = lax.dot_general(xb, w1_ref[0], (((1,), (1,)), ((), ())),
                            preferred_element_type=jnp.float32)
        h = h + b1_ref[0]
        h = 0.5 * h * (1.0 + lax.erf(h * jnp.float32(0.7071067811865476)))
        y = lax.dot_general(h, w2_ref[0], (((1,), (1,)), ((), ())),
                            preferred_element_type=jnp.float32)
        out_ref[...] = y + b2_ref[0]


def _ffn(exp_id, valid, xg, W1, b1, W2, b2):
    grid_spec = pltpu.PrefetchScalarGridSpec(
        num_scalar_prefetch=2,
        grid=(NB,),
        in_specs=[
            pl.BlockSpec((BLK, H), lambda b, eref, vref: (b, 0)),
            pl.BlockSpec((1, DFF, H), lambda b, eref, vref: (eref[b], 0, 0)),
            pl.BlockSpec((1, 1, DFF), lambda b, eref, vref: (eref[b], 0, 0)),
            pl.BlockSpec((1, H, DFF), lambda b, eref, vref: (eref[b], 0, 0)),
            pl.BlockSpec((1, 1, H), lambda b, eref, vref: (eref[b], 0, 0)),
        ],
        out_specs=pl.BlockSpec((BLK, H), lambda b, eref, vref: (b, 0)),
    )
    return pl.pallas_call(
        _ffn_body,
        grid_spec=grid_spec,
        out_shape=jax.ShapeDtypeStruct((NROWS, H), jnp.float32),
    )(exp_id, valid, xg, W1, b1.reshape(E, 1, DFF), W2, b2.reshape(E, 1, H))


def _combine_kernel(d1_hbm, d2_hbm, ybuf_hbm, yab_hbm, idxr, rows, sem):
    wid = lax.axis_index("s") * NC + lax.axis_index("c")
    t0 = wid * TPT

    pltpu.sync_copy(d1_hbm.at[pl.ds(t0, TPT)], idxr)
    pltpu.async_copy(ybuf_hbm.at[idxr], rows, sem).wait()
    pltpu.sync_copy(rows, yab_hbm.at[pl.ds(t0, TPT)])

    pltpu.sync_copy(d2_hbm.at[pl.ds(t0, TPT)], idxr)
    pltpu.async_copy(ybuf_hbm.at[idxr], rows, sem).wait()
    pltpu.sync_copy(rows, yab_hbm.at[pl.ds(S + t0, TPT)])


def _combine(d1, d2, ybuf):
    mesh = plsc.VectorSubcoreMesh(core_axis_name="c", subcore_axis_name="s")
    return pl.kernel(
        _combine_kernel,
        mesh=mesh,
        compiler_params=pltpu.CompilerParams(needs_layout_passes=False),
        out_type=jax.ShapeDtypeStruct((2 * S, H), jnp.float32),
        scratch_types=[
            pltpu.VMEM((TPT,), jnp.int32),
            pltpu.VMEM((TPT, H), jnp.float32),
            pltpu.SemaphoreType.DMA,
        ],
    )(d1, d2, ybuf)


def _add_body(a_ref, b_ref, ga_ref, gb_ref, out_ref):
    out_ref[...] = a_ref[...] * ga_ref[...] + b_ref[...] * gb_ref[...]


def _add(yab, gcat):
    nsb = 4
    sb = S // nsb
    return pl.pallas_call(
        _add_body,
        grid=(nsb,),
        out_shape=jax.ShapeDtypeStruct((S, H), jnp.float32),
        in_specs=[
            pl.BlockSpec((sb, H), lambda i: (i, 0)),
            pl.BlockSpec((sb, H), lambda i: (i + nsb, 0)),
            pl.BlockSpec((sb, 1), lambda i: (i, 0)),
            pl.BlockSpec((sb, 1), lambda i: (i + nsb, 0)),
        ],
        out_specs=pl.BlockSpec((sb, H), lambda i: (i, 0)),
    )(yab, yab, gcat, gcat)


@jax.jit
def _moe(x, Wr, W1, b1, W2, b2):
    x2d = x.reshape(S, H)
    wr_pad = jnp.zeros((EPAD, H), jnp.float32).at[:E].set(Wr)

    d1c, d2c, g1c, g2c, counts_row, aux = _routing(x2d, wr_pad)
    d1 = d1c.reshape(S)
    d2 = d2c.reshape(S)
    gcat = jnp.concatenate([g1c, g2c], axis=0)

    counts = counts_row[0, :E].astype(jnp.int32)
    nb = (counts + (BLK - 1)) // BLK
    cum = jnp.cumsum(nb)
    barange = jnp.arange(NB, dtype=jnp.int32)
    exp_id = jnp.minimum(
        jnp.sum((barange[:, None] >= cum[None, :]).astype(jnp.int32), axis=1),
        E - 1)
    valid = (barange < cum[-1]).astype(jnp.int32)

    xg = _dispatch(d1, d2, x2d)
    ybuf = _ffn(exp_id, valid, xg, W1, b1, W2, b2)
    yab = _combine(d1, d2, ybuf)
    out = _add(yab, gcat)
    return out.reshape(1, S, H), aux[0, 0]


def kernel(x, Wr, W1, b1, W2, b2):
    return _moe(x, Wr, W1, b1, W2, b2)
